# baseline (speedup 1.0000x reference)
import functools

import jax
import jax.numpy as jnp
from jax import lax
from jax.experimental import pallas as pl
from jax.experimental.pallas import tpu as pltpu
from jax.experimental.pallas import tpu_sc as plsc

_C = 256


def _tc_body(t_ref, i_ref, a_ref, emb_ref, out_ref, *, n_t, n_i):
    j = pl.program_id(1)

    @pl.when(j < n_t)
    def _():
        out_ref[...] = t_ref[...] + emb_ref[0, :][None, None, :]

    @pl.when((j >= n_t) & (j < n_t + n_i))
    def _():
        out_ref[...] = i_ref[...] + emb_ref[1, :][None, None, :]

    @pl.when(j >= n_t + n_i)
    def _():
        out_ref[...] = a_ref[...] + emb_ref[2, :][None, None, :]


def _sc_ids_body(ids_hbm, ids_v, *, B, seg_lens, tot, nw, nc):
    cid = lax.axis_index("c")
    sid = lax.axis_index("s")
    wid = sid * nc + cid

    off = 0
    for m, lm in enumerate(seg_lens):
        rm = B * lm // nw
        base = wid * rm
        b = base // lm
        col0 = off + (base - b * lm)
        ids_off = sum(B * l // nw for l in seg_lens[:m])
        for i in range(rm // 16):
            ids_v[pl.ds(ids_off + i * 16, 16)] = jnp.full((16,), m, jnp.int32)
        pltpu.sync_copy(ids_v.at[pl.ds(ids_off, rm)],
                        ids_hbm.at[b, pl.ds(col0, rm)])
        off += lm


def kernel(text, image, audio, modality_embed):
    B, l_t, H = text.shape
    l_i = image.shape[1]
    l_a = audio.shape[1]
    tot = l_t + l_i + l_a
    n_t, n_i, n_a = l_t // _C, l_i // _C, l_a // _C

    info = plsc.get_sparse_core_info()
    nc, ns = info.num_cores, info.num_subcores
    nw = nc * ns
    mesh = plsc.VectorSubcoreMesh(core_axis_name="c", subcore_axis_name="s")

    ids = pl.kernel(
        functools.partial(_sc_ids_body, B=B, seg_lens=(l_t, l_i, l_a),
                          tot=tot, nw=nw, nc=nc),
        mesh=mesh,
        out_type=[jax.ShapeDtypeStruct((B, tot), jnp.int32)],
        scratch_types=[pltpu.VMEM((B * tot // nw,), jnp.int32)],
    )()[0]

    out = pl.pallas_call(
        functools.partial(_tc_body, n_t=n_t, n_i=n_i),
        grid=(B, n_t + n_i + n_a),
        in_specs=[
            pl.BlockSpec((1, _C, H), lambda b, j: (b, jnp.minimum(j, n_t - 1), 0)),
            pl.BlockSpec((1, _C, H), lambda b, j: (b, jnp.clip(j - n_t, 0, n_i - 1), 0)),
            pl.BlockSpec((1, _C, H), lambda b, j: (b, jnp.clip(j - n_t - n_i, 0, n_a - 1), 0)),
            pl.BlockSpec(modality_embed.shape, lambda b, j: (0, 0)),
        ],
        out_specs=pl.BlockSpec((1, _C, H), lambda b, j: (b, j, 0)),
        out_shape=jax.ShapeDtypeStruct((B, tot, H), jnp.float32),
    )(text, image, audio, modality_embed)

    return out, ids


# speedup vs baseline: 1.0856x; 1.0856x over previous
import functools

import jax
import jax.numpy as jnp
from jax import lax
from jax.experimental import pallas as pl
from jax.experimental.pallas import tpu as pltpu
from jax.experimental.pallas import tpu_sc as plsc

_C = 512


def _tc_body(t_ref, i_ref, a_ref, emb_ref, out_ref, *, n_t, n_i):
    j = pl.program_id(1)

    @pl.when(j < n_t)
    def _():
        out_ref[...] = t_ref[...] + emb_ref[0, :][None, None, :]

    @pl.when((j >= n_t) & (j < n_t + n_i))
    def _():
        out_ref[...] = i_ref[...] + emb_ref[1, :][None, None, :]

    @pl.when(j >= n_t + n_i)
    def _():
        out_ref[...] = a_ref[...] + emb_ref[2, :][None, None, :]


def _sc_ids_body(ids_hbm, ids_v, *, B, seg_lens, tot, nw, nc):
    cid = lax.axis_index("c")
    sid = lax.axis_index("s")
    wid = sid * nc + cid

    off = 0
    for m, lm in enumerate(seg_lens):
        rm = B * lm // nw
        base = wid * rm
        b = base // lm
        col0 = off + (base - b * lm)
        ids_off = sum(B * l // nw for l in seg_lens[:m])
        for i in range(rm // 16):
            ids_v[pl.ds(ids_off + i * 16, 16)] = jnp.full((16,), m, jnp.int32)
        pltpu.sync_copy(ids_v.at[pl.ds(ids_off, rm)],
                        ids_hbm.at[b, pl.ds(col0, rm)])
        off += lm


def kernel(text, image, audio, modality_embed):
    B, l_t, H = text.shape
    l_i = image.shape[1]
    l_a = audio.shape[1]
    tot = l_t + l_i + l_a
    n_t, n_i, n_a = l_t // _C, l_i // _C, l_a // _C

    info = plsc.get_sparse_core_info()
    nc, ns = info.num_cores, info.num_subcores
    nw = nc * ns
    mesh = plsc.VectorSubcoreMesh(core_axis_name="c", subcore_axis_name="s")

    ids = pl.kernel(
        functools.partial(_sc_ids_body, B=B, seg_lens=(l_t, l_i, l_a),
                          tot=tot, nw=nw, nc=nc),
        mesh=mesh,
        out_type=[jax.ShapeDtypeStruct((B, tot), jnp.int32)],
        scratch_types=[pltpu.VMEM((B * tot // nw,), jnp.int32)],
    )()[0]

    out = pl.pallas_call(
        functools.partial(_tc_body, n_t=n_t, n_i=n_i),
        grid=(B, n_t + n_i + n_a),
        in_specs=[
            pl.BlockSpec((1, _C, H), lambda b, j: (b, jnp.minimum(j, n_t - 1), 0)),
            pl.BlockSpec((1, _C, H), lambda b, j: (b, jnp.clip(j - n_t, 0, n_i - 1), 0)),
            pl.BlockSpec((1, _C, H), lambda b, j: (b, jnp.clip(j - n_t - n_i, 0, n_a - 1), 0)),
            pl.BlockSpec(modality_embed.shape, lambda b, j: (0, 0)),
        ],
        out_specs=pl.BlockSpec((1, _C, H), lambda b, j: (b, j, 0)),
        out_shape=jax.ShapeDtypeStruct((B, tot, H), jnp.float32),
    )(text, image, audio, modality_embed)

    return out, ids
